# baseline (speedup 1.0000x reference)
import functools

import jax
import jax.numpy as jnp
from jax import lax
from jax.experimental import pallas as pl
from jax.experimental.pallas import tpu as pltpu
from jax.experimental.pallas import tpu_sc as plsc

N = 100000
D = 64
B = 16384
L = 16
NC = 2
NS = 16
NW = NC * NS
BPW = B // NW
NVEC = B // L
CHUNK = 128
NCHUNK = BPW // CHUNK

_mesh = plsc.VectorSubcoreMesh(core_axis_name="c", subcore_axis_name="s")


@functools.partial(
    pl.kernel,
    out_type=tuple(jax.ShapeDtypeStruct((B, D), jnp.float32) for _ in range(5)),
    mesh=_mesh,
    compiler_params=pltpu.CompilerParams(
        needs_layout_passes=False, use_tc_tiling_on_sc=False),
    scratch_types=[
        pltpu.VMEM((N,), jnp.int32),
        pltpu.VMEM((B // 4,), jnp.int32),
        pltpu.VMEM((L,), jnp.int32),
        pltpu.VMEM((NCHUNK, CHUNK), jnp.int32),
        pltpu.VMEM((NCHUNK, CHUNK), jnp.int32),
        pltpu.VMEM((CHUNK, D), jnp.float32),
        pltpu.VMEM((CHUNK, D), jnp.float32),
        pltpu.VMEM_SHARED((N,), jnp.int32),
        pltpu.SemaphoreType.DMA,
        pltpu.SemaphoreType.DMA,
        pltpu.SemaphoreType.DMA,
        pltpu.SemaphoreType.DMA,
    ],
)
def _memory_kernel(idx_hbm, v1, v2, v3, v4, v5, o1, o2, o3, o4, o5,
                   pos, xm, scr, idx_v, m_v, buf_a, buf_b, pos_sh,
                   gsem_a, gsem_b, wsem_a, wsem_b):
    core = lax.axis_index("c")
    sub = lax.axis_index("s")
    wid = sub * NC + core
    base = wid * BPW

    @pl.when(sub == 0)
    def _():
        lane = lax.iota(jnp.int32, L)
        nxt_lane = jnp.minimum(lane + 1, L - 1)
        is_last_lane = lane == L - 1
        seg = B // 4
        seg_vecs = seg // L

        for g in range(4):
            pltpu.sync_copy(idx_hbm.at[pl.ds(g * seg, seg)], xm)

            def scatter_body(c, carry, g=g):
                x = xm[pl.ds(c * L, L)]
                comp = (x << 14) | (lane + (g * seg_vecs + c) * L)
                s, _ = plsc.sort_key_val(comp, comp)
                scr[...] = s
                s_nxt = plsc.load_gather(scr, [nxt_lane])
                keep = ((s >> 14) != (s_nxt >> 14)) | is_last_lane
                plsc.store_scatter(pos, [s >> 14], s & 16383, mask=keep)
                return carry

            lax.fori_loop(0, seg_vecs, scatter_body, 0)
        pltpu.sync_copy(pos, pos_sh)

    plsc.subcore_barrier()

    for j in range(NCHUNK):
        pltpu.sync_copy(idx_hbm.at[pl.ds(base + j * CHUNK, CHUNK)],
                        idx_v.at[j])
    mcp = [pltpu.async_copy(pos_sh.at[idx_v.at[j]], m_v.at[j], gsem_a)
           for j in range(NCHUNK)]
    for cp in mcp:
        cp.wait()

    vs = (v1, v2, v3, v4, v5)
    os_ = (o1, o2, o3, o4, o5)
    bufs = (buf_a, buf_b)
    gsems = (gsem_a, gsem_b)
    wsems = (wsem_a, wsem_b)
    steps = [(k, j) for k in range(5) for j in range(NCHUNK)]

    def fire(t):
        k, j = steps[t]
        return pltpu.async_copy(vs[k].at[m_v.at[j]], bufs[t % 2],
                                gsems[t % 2])

    wcp = [None, None]
    cp = fire(0)
    for t in range(len(steps)):
        if t + 1 < len(steps):
            if wcp[(t + 1) % 2] is not None:
                wcp[(t + 1) % 2].wait()
            nxt = fire(t + 1)
        else:
            nxt = None
        cp.wait()
        k, j = steps[t]
        wcp[t % 2] = pltpu.async_copy(
            bufs[t % 2], os_[k].at[pl.ds(base + j * CHUNK, CHUNK)],
            wsems[t % 2])
        cp = nxt
    for w in wcp:
        if w is not None:
            w.wait()


def kernel(nodes_memory, crowds_memory, interests_memory, categories_memory,
           brands_memory, values1, values2, values3, values4, values5,
           users_idxs):
    return _memory_kernel(users_idxs, values1, values2, values3, values4,
                          values5)
